# Initial kernel scaffold; baseline (speedup 1.0000x reference)
import functools

import jax
import jax.numpy as jnp
from jax import lax
from jax.experimental import pallas as pl
from jax.experimental.pallas import tpu as pltpu
from jax.experimental.pallas import tpu_sc as plsc

D = 64
SEQ = 200
NW = 32
CHUNK = 128
ROWS_W = 25600
NCHUNK = ROWS_W // CHUNK
NBUF = 4
LANES = 16


def _sc_body(x2_hbm, table_hbm, pos2_hbm, out_hbm, *scratch):
    idx_v = scratch[0]
    pos_v = scratch[1]
    gbufs = scratch[2:2 + NBUF]
    sbufs = scratch[2 + NBUF:2 + 2 * NBUF]
    gsems = scratch[2 + 2 * NBUF:2 + 3 * NBUF]
    ssems = scratch[2 + 3 * NBUF:2 + 4 * NBUF]

    wid = lax.axis_index("c") * 16 + lax.axis_index("s")
    base_row = wid * ROWS_W

    pltpu.sync_copy(x2_hbm.at[pl.ds(wid * NCHUNK, NCHUNK), :], idx_v)
    pltpu.sync_copy(pos2_hbm, pos_v)

    def gather(g, b):
        return pltpu.make_async_copy(
            table_hbm.at[idx_v.at[g]], gbufs[b], gsems[b])

    def scatter(g, b):
        return pltpu.make_async_copy(
            sbufs[b], out_hbm.at[pl.ds(base_row + g * CHUNK, CHUNK), :],
            ssems[b])

    for b in range(NBUF):
        gather(b, b).start()

    def outer(t, carry):
        for b in range(NBUF):
            g = t * NBUF + b
            gather(g, b).wait()

            @pl.when(t > 0)
            def _():
                scatter(g - NBUF, b).wait()

            p = lax.rem(g * CHUNK, SEQ)

            def add_row(r, _, b=b, p=p):
                for j in range(D // LANES):
                    sl = pl.ds(j * LANES, LANES)
                    sbufs[b][r, sl] = gbufs[b][r, sl] + pos_v[p + r, sl]
                return 0

            lax.fori_loop(0, CHUNK, add_row, 0)

            scatter(g, b).start()

            @pl.when(g + NBUF < NCHUNK)
            def _():
                gather(g + NBUF, b).start()
        return carry

    lax.fori_loop(0, NCHUNK // NBUF, outer, 0)

    for b in range(NBUF):
        scatter(NCHUNK - NBUF + b, b).wait()


_scratch = (
    [pltpu.VMEM((NCHUNK, CHUNK), jnp.int32),
     pltpu.VMEM((2 * SEQ, D), jnp.float32)]
    + [pltpu.VMEM((CHUNK, D), jnp.float32) for _ in range(2 * NBUF)]
    + [pltpu.SemaphoreType.DMA for _ in range(2 * NBUF)]
)

_sc_embed = functools.partial(
    pl.kernel,
    out_type=jax.ShapeDtypeStruct((NW * ROWS_W, D), jnp.float32),
    mesh=plsc.VectorSubcoreMesh(core_axis_name="c", subcore_axis_name="s"),
    scratch_types=_scratch,
)(_sc_body)


def kernel(x, table, pos_embedding):
    B, N = x.shape
    x2 = x.astype(jnp.int32).reshape(-1, CHUNK)
    posf = pos_embedding[0, :N, :]
    pos2 = jnp.concatenate([posf, posf], axis=0)
    out = _sc_embed(x2, table, pos2)
    return out.reshape(B, N, D)


# speedup vs baseline: 2.7251x; 2.7251x over previous
import functools

import jax
import jax.numpy as jnp
from jax import lax
from jax.experimental import pallas as pl
from jax.experimental.pallas import tpu as pltpu
from jax.experimental.pallas import tpu_sc as plsc

D = 64
SEQ = 200
NW = 32
CHUNK = 128
ROWS_W = 25600
NCHUNK = ROWS_W // CHUNK
NBUF = 4
LANES = 16


def _sc_body(x2_hbm, table_hbm, pos2_hbm, out_hbm, *scratch):
    idx_v = scratch[0]
    pos_v = scratch[1]
    gbufs = scratch[2:2 + NBUF]
    sbufs = scratch[2 + NBUF:2 + 2 * NBUF]
    gsems = scratch[2 + 2 * NBUF:2 + 3 * NBUF]
    ssems = scratch[2 + 3 * NBUF:2 + 4 * NBUF]

    wid = lax.axis_index("c") * 16 + lax.axis_index("s")
    base_row = wid * ROWS_W

    pltpu.sync_copy(x2_hbm.at[pl.ds(wid * NCHUNK, NCHUNK), :], idx_v)
    pltpu.sync_copy(pos2_hbm, pos_v)

    def gather(g, b):
        return pltpu.make_async_copy(
            table_hbm.at[idx_v.at[g]], gbufs[b], gsems[b])

    def scatter(g, b):
        return pltpu.make_async_copy(
            sbufs[b], out_hbm.at[pl.ds(base_row + g * CHUNK, CHUNK), :],
            ssems[b])

    for b in range(NBUF):
        gather(b, b).start()

    def outer(t, carry):
        for b in range(NBUF):
            g = t * NBUF + b
            gather(g, b).wait()

            @pl.when(t > 0)
            def _():
                scatter(g - NBUF, b).wait()

            p = lax.rem(g * CHUNK, SEQ)

            def add_row(r, _, b=b, p=p):
                for j in range(D // LANES):
                    sl = pl.ds(j * LANES, LANES)
                    sbufs[b][r, sl] = gbufs[b][r, sl] + pos_v[p + r, sl]
                return 0

            lax.fori_loop(0, CHUNK, add_row, 0)

            scatter(g, b).start()

            @pl.when(g + NBUF < NCHUNK)
            def _():
                gather(g + NBUF, b).start()
        return carry

    lax.fori_loop(0, NCHUNK // NBUF, outer, 0)

    for b in range(NBUF):
        scatter(NCHUNK - NBUF + b, b).wait()


_scratch = (
    [pltpu.VMEM((NCHUNK, CHUNK), jnp.int32),
     pltpu.VMEM((2 * SEQ, D), jnp.float32)]
    + [pltpu.VMEM((CHUNK, D), jnp.float32) for _ in range(2 * NBUF)]
    + [pltpu.SemaphoreType.DMA for _ in range(2 * NBUF)]
)

_sc_embed = functools.partial(
    pl.kernel,
    out_type=jax.ShapeDtypeStruct((NW * ROWS_W, D), jnp.float32),
    mesh=plsc.VectorSubcoreMesh(core_axis_name="c", subcore_axis_name="s"),
    scratch_types=_scratch,
    compiler_params=pltpu.CompilerParams(use_tc_tiling_on_sc=False),
)(_sc_body)


def kernel(x, table, pos_embedding):
    B, N = x.shape
    x2 = x.astype(jnp.int32).reshape(-1, CHUNK)
    posf = pos_embedding[0, :N, :]
    pos2 = jnp.concatenate([posf, posf], axis=0)
    out = _sc_embed(x2, table, pos2)
    return out.reshape(B, N, D)


# speedup vs baseline: 2.7439x; 1.0069x over previous
import functools

import jax
import jax.numpy as jnp
from jax import lax
from jax.experimental import pallas as pl
from jax.experimental.pallas import tpu as pltpu
from jax.experimental.pallas import tpu_sc as plsc

D = 64
SEQ = 200
NW = 32
CHUNK = 128
ROWS_W = 25600
NCHUNK = ROWS_W // CHUNK
NBUF = 4
LANES = 16


def _sc_body(x2_hbm, table_hbm, pos2_hbm, out_hbm, *scratch):
    idx_v = scratch[0]
    pos_v = scratch[1]
    gbufs = scratch[2:2 + NBUF]
    sbufs = scratch[2 + NBUF:2 + 2 * NBUF]
    gsems = scratch[2 + 2 * NBUF:2 + 3 * NBUF]
    ssems = scratch[2 + 3 * NBUF:2 + 4 * NBUF]

    wid = lax.axis_index("c") * 16 + lax.axis_index("s")
    base2 = wid * (ROWS_W // 2)

    pltpu.sync_copy(x2_hbm.at[pl.ds(wid * NCHUNK, NCHUNK), :], idx_v)
    pltpu.sync_copy(pos2_hbm, pos_v)

    def gather(g, b):
        return pltpu.make_async_copy(
            table_hbm.at[idx_v.at[g]], gbufs[b], gsems[b])

    def scatter(g, b):
        return pltpu.make_async_copy(
            sbufs[b],
            out_hbm.at[pl.ds(base2 + g * (CHUNK // 2), CHUNK // 2), :],
            ssems[b])

    for b in range(NBUF):
        gather(b, b).start()

    def outer(t, carry):
        for b in range(NBUF):
            g = t * NBUF + b
            gather(g, b).wait()

            @pl.when(t > 0)
            def _():
                scatter(g - NBUF, b).wait()

            p2 = lax.rem(g * (CHUNK // 2), SEQ // 2)

            def add_row(pr, _, b=b, p2=p2):
                for h in range(2):
                    for j in range(D // LANES):
                        sl = pl.ds(h * D + j * LANES, LANES)
                        sbufs[b][pr, sl] = (
                            gbufs[b][2 * pr + h, pl.ds(j * LANES, LANES)]
                            + pos_v[p2 + pr, sl])
                return 0

            lax.fori_loop(0, CHUNK // 2, add_row, 0)

            scatter(g, b).start()

            @pl.when(g + NBUF < NCHUNK)
            def _():
                gather(g + NBUF, b).start()
        return carry

    lax.fori_loop(0, NCHUNK // NBUF, outer, 0)

    for b in range(NBUF):
        scatter(NCHUNK - NBUF + b, b).wait()


_scratch = (
    [pltpu.VMEM((NCHUNK, CHUNK), jnp.int32),
     pltpu.VMEM((SEQ, 2 * D), jnp.float32)]
    + [pltpu.VMEM((CHUNK, D), jnp.float32) for _ in range(NBUF)]
    + [pltpu.VMEM((CHUNK // 2, 2 * D), jnp.float32) for _ in range(NBUF)]
    + [pltpu.SemaphoreType.DMA for _ in range(2 * NBUF)]
)

_sc_embed = functools.partial(
    pl.kernel,
    out_type=jax.ShapeDtypeStruct((NW * ROWS_W // 2, 2 * D), jnp.float32),
    mesh=plsc.VectorSubcoreMesh(core_axis_name="c", subcore_axis_name="s"),
    scratch_types=_scratch,
    compiler_params=pltpu.CompilerParams(use_tc_tiling_on_sc=False),
)(_sc_body)


def kernel(x, table, pos_embedding):
    B, N = x.shape
    x2 = x.astype(jnp.int32).reshape(-1, CHUNK)
    posf = pos_embedding[0, :N, :]
    pos2 = jnp.concatenate([posf, posf], axis=0).reshape(SEQ, 2 * D)
    out = _sc_embed(x2, table, pos2)
    return out.reshape(B, N, D)
